# baseline (speedup 1.0000x reference)
import functools

import jax
import jax.numpy as jnp
from jax.experimental import pallas as pl
from jax.experimental.pallas import tpu as pltpu

EPS = 1e-12


def _body(x_ref, p_ref, w_ref, b_ref, g_ref, be_ref, o_ref):
    x = (x_ref[0] + p_ref[...]).astype(jnp.bfloat16)
    y = jnp.dot(x, w_ref[...], preferred_element_type=jnp.float32)
    bb = b_ref[...]
    nd = 1.0 / y.shape[-1]
    b_sum = jnp.sum(bb) * nd
    b2_sum = jnp.sum(bb * bb) * nd
    m1 = jnp.sum(y, axis=-1, keepdims=True) * nd + b_sum
    q = jnp.sum(y * (y + 2.0 * bb), axis=-1, keepdims=True) * nd + b2_sum
    r = jax.lax.rsqrt(q - m1 * m1 + EPS)
    o_ref[0] = (y + (bb - m1)) * r * g_ref[...] + be_ref[...]


@functools.partial(jax.jit, static_argnames=())
def kernel(input_tensor, pos_table, W, b, gamma, beta):
    B, S, D = input_tensor.shape
    DH = W.shape[1]
    BM = min(1024, S)
    n_s = S // BM

    grid = (n_s, B)
    out = pl.pallas_call(
        _body,
        grid=grid,
        in_specs=[
            pl.BlockSpec((1, BM, D), lambda s, bi: (bi, s, 0)),
            pl.BlockSpec((BM, D), lambda s, bi: (s, 0)),
            pl.BlockSpec((D, DH), lambda s, bi: (0, 0)),
            pl.BlockSpec((1, DH), lambda s, bi: (0, 0)),
            pl.BlockSpec((1, DH), lambda s, bi: (0, 0)),
            pl.BlockSpec((1, DH), lambda s, bi: (0, 0)),
        ],
        out_specs=pl.BlockSpec((1, BM, DH), lambda s, bi: (bi, s, 0)),
        out_shape=jax.ShapeDtypeStruct((B, S, DH), jnp.float32),
        compiler_params=pltpu.CompilerParams(
            dimension_semantics=("parallel", "parallel"),
        ),
    )(
        input_tensor,
        pos_table,
        W.astype(jnp.bfloat16),
        b.reshape(1, DH),
        gamma.reshape(1, DH),
        beta.reshape(1, DH),
    )
    return out


# speedup vs baseline: 1.0070x; 1.0070x over previous
import functools

import jax
import jax.numpy as jnp
from jax.experimental import pallas as pl
from jax.experimental.pallas import tpu as pltpu

EPS = 1e-12


def _body(x_ref, p_ref, w_ref, b_ref, g_ref, be_ref, o_ref):
    x = (x_ref[0] + p_ref[...]).astype(jnp.bfloat16)
    y = jnp.dot(x, w_ref[...], preferred_element_type=jnp.float32)
    o_ref[0] = y + b_ref[...]
    t = o_ref[0]
    m1 = jnp.mean(t, axis=-1, keepdims=True)
    m2 = jnp.mean(t * t, axis=-1, keepdims=True)
    r = jax.lax.rsqrt(m2 - m1 * m1 + EPS)
    o_ref[0] = (t - m1) * r * g_ref[...] + be_ref[...]


@functools.partial(jax.jit, static_argnames=())
def kernel(input_tensor, pos_table, W, b, gamma, beta):
    B, S, D = input_tensor.shape
    DH = W.shape[1]
    BM = min(1024, S)
    n_s = S // BM

    grid = (n_s, B)
    out = pl.pallas_call(
        _body,
        grid=grid,
        in_specs=[
            pl.BlockSpec((1, BM, D), lambda s, bi: (bi, s, 0)),
            pl.BlockSpec((BM, D), lambda s, bi: (s, 0)),
            pl.BlockSpec((D, DH), lambda s, bi: (0, 0)),
            pl.BlockSpec((1, DH), lambda s, bi: (0, 0)),
            pl.BlockSpec((1, DH), lambda s, bi: (0, 0)),
            pl.BlockSpec((1, DH), lambda s, bi: (0, 0)),
        ],
        out_specs=pl.BlockSpec((1, BM, DH), lambda s, bi: (bi, s, 0)),
        out_shape=jax.ShapeDtypeStruct((B, S, DH), jnp.float32),
        compiler_params=pltpu.CompilerParams(
            dimension_semantics=("parallel", "parallel"),
        ),
    )(
        input_tensor,
        pos_table,
        W.astype(jnp.bfloat16),
        b.reshape(1, DH),
        gamma.reshape(1, DH),
        beta.reshape(1, DH),
    )
    return out


# speedup vs baseline: 1.0388x; 1.0315x over previous
import functools

import jax
import jax.numpy as jnp
from jax.experimental import pallas as pl
from jax.experimental.pallas import tpu as pltpu

EPS = 1e-12


def _body(x_ref, p_ref, w_ref, b_ref, g_ref, be_ref, o_ref):
    x = (x_ref[0] + p_ref[...]).astype(jnp.bfloat16)
    y = jnp.dot(x, w_ref[...], preferred_element_type=jnp.float32)
    o_ref[0] = y + b_ref[...]
    t = o_ref[0]
    m1 = jnp.mean(t, axis=-1, keepdims=True)
    m2 = jnp.mean(t * t, axis=-1, keepdims=True)
    r = jax.lax.rsqrt(m2 - m1 * m1 + EPS)
    o_ref[0] = (t - m1) * r * g_ref[...] + be_ref[...]


@functools.partial(jax.jit, static_argnames=())
def kernel(input_tensor, pos_table, W, b, gamma, beta):
    B, S, D = input_tensor.shape
    DH = W.shape[1]
    BM = min(2048, S)
    n_s = S // BM

    grid = (n_s, B)
    out = pl.pallas_call(
        _body,
        grid=grid,
        in_specs=[
            pl.BlockSpec((1, BM, D), lambda s, bi: (bi, s, 0)),
            pl.BlockSpec((BM, D), lambda s, bi: (s, 0)),
            pl.BlockSpec((D, DH), lambda s, bi: (0, 0)),
            pl.BlockSpec((1, DH), lambda s, bi: (0, 0)),
            pl.BlockSpec((1, DH), lambda s, bi: (0, 0)),
            pl.BlockSpec((1, DH), lambda s, bi: (0, 0)),
        ],
        out_specs=pl.BlockSpec((1, BM, DH), lambda s, bi: (bi, s, 0)),
        out_shape=jax.ShapeDtypeStruct((B, S, DH), jnp.float32),
        compiler_params=pltpu.CompilerParams(
            dimension_semantics=("parallel", "parallel"),
        ),
    )(
        input_tensor,
        pos_table,
        W.astype(jnp.bfloat16),
        b.reshape(1, DH),
        gamma.reshape(1, DH),
        beta.reshape(1, DH),
    )
    return out
